# baseline (speedup 1.0000x reference)
import functools

import jax
import jax.numpy as jnp
from jax import lax
from jax.experimental import pallas as pl
from jax.experimental.pallas import tpu as pltpu
from jax.experimental.pallas import tpu_sc as plsc

N = 10000
D = 128
E = 320000
C = 40
NC = 2
NS = 16
NW = NC * NS
CHUNK = 128
EPAD = 327680
CT = EPAD // (NW * CHUNK)
HCT = 32
SLOW_CORE = 0
CT0 = 32
CT1 = 128
NACC = 10112
RT = NACC // NS
EW = 16

_QCHUNKS = [(q * CHUNK, CHUNK) for q in range(RT // CHUNK)]
if RT % CHUNK:
    _QCHUNKS.append((RT - RT % CHUNK, RT % CHUNK))


def _sc_count_body(consts, dst2d, c_out, dst_v, buf_v, cnt_sh):
    c = lax.axis_index("c")
    s = lax.axis_index("s")
    wid = c * NS + s

    pltpu.sync_copy(consts.at[0], buf_v)
    for off, rows in _QCHUNKS:
        pltpu.sync_copy(buf_v.at[pl.ds(0, rows)],
                        cnt_sh.at[pl.ds(s * RT + off, rows)])

    pltpu.sync_copy(consts.at[1], buf_v)
    plsc.subcore_barrier()

    pltpu.sync_copy(dst2d.at[pl.ds(wid * CT, CT)], dst_v)

    @pl.loop(0, CT)
    def _edge_chunk(j):
        pltpu.sync_copy(buf_v, cnt_sh.at[dst_v.at[j]], add=True)

    plsc.subcore_barrier()

    for off, rows in _QCHUNKS:
        b = s * RT + off
        pltpu.sync_copy(cnt_sh.at[pl.ds(b, rows)], buf_v.at[pl.ds(0, rows)])
        pltpu.sync_copy(buf_v.at[pl.ds(0, rows)], c_out.at[c, pl.ds(b, rows)])


@functools.lru_cache(maxsize=None)
def _sc_count_kernel():
    mesh = plsc.VectorSubcoreMesh(
        core_axis_name="c", subcore_axis_name="s",
        num_cores=NC, num_subcores=NS)
    return pl.kernel(
        _sc_count_body,
        out_type=jax.ShapeDtypeStruct((NC, NACC, D), jnp.float32),
        mesh=mesh,
        scratch_types=[
            pltpu.VMEM((CT, CHUNK), jnp.int32),
            pltpu.VMEM((CHUNK, D), jnp.float32),
            pltpu.VMEM_SHARED((NACC, D), jnp.float32),
        ],
        name="sc_sage_count",
    )


def _sc_agg_body(width, consts, table, src2d, dst2d, s_out, src_v, dst_v,
                 rows_v, rows_w, acc_sh, sem_a, sem_b):
    c = lax.axis_index("c")
    s = lax.axis_index("s")
    wid = c * NS + s

    pltpu.sync_copy(consts.at[0], rows_v)
    for off, rows in _QCHUNKS:
        pltpu.sync_copy(rows_v.at[pl.ds(0, rows)],
                        acc_sh.at[pl.ds(s * RT + off, rows)])

    plsc.subcore_barrier()

    bufs = (rows_v, rows_w)
    sems = (sem_a, sem_b)

    def _pipeline(base, n):
        pltpu.sync_copy(src2d.at[pl.ds(base, n)], src_v)
        pltpu.sync_copy(dst2d.at[pl.ds(base, n)], dst_v)
        cp = [None] * n
        cp[0] = pltpu.async_copy(table.at[src_v.at[0]], bufs[0], sems[0])
        if n > 1:
            cp[1] = pltpu.async_copy(table.at[src_v.at[1]], bufs[1], sems[1])
        for j in range(n):
            p = j % 2
            cp[j].wait()
            pltpu.sync_copy(bufs[p], acc_sh.at[dst_v.at[j]], add=True)
            if j + 2 < n:
                cp[j + 2] = pltpu.async_copy(
                    table.at[src_v.at[j + 2]], bufs[p], sems[p])

    @pl.when(c == SLOW_CORE)
    def _slow_core_edges():
        for st in range(CT0 // HCT):
            _pipeline(s * CT0 + st * HCT, HCT)

    @pl.when(c != SLOW_CORE)
    def _fast_core_edges():
        base0 = NS * CT0 + s * CT1
        for st in range(CT1 // HCT):
            _pipeline(base0 + st * HCT, HCT)

    plsc.subcore_barrier()

    for off, rows in _QCHUNKS:
        b = s * RT + off
        pltpu.sync_copy(acc_sh.at[pl.ds(b, rows)], rows_v.at[pl.ds(0, rows)])
        pltpu.sync_copy(rows_v.at[pl.ds(0, rows)], s_out.at[c, pl.ds(b, rows)])


@functools.lru_cache(maxsize=None)
def _sc_agg_kernel(width):
    mesh = plsc.VectorSubcoreMesh(
        core_axis_name="c", subcore_axis_name="s",
        num_cores=NC, num_subcores=NS)
    return pl.kernel(
        functools.partial(_sc_agg_body, width),
        out_type=jax.ShapeDtypeStruct((NC, NACC, width), jnp.float32),
        mesh=mesh,
        scratch_types=[
            pltpu.VMEM((HCT, CHUNK), jnp.int32),
            pltpu.VMEM((HCT, CHUNK), jnp.int32),
            pltpu.VMEM((CHUNK, width), jnp.float32),
            pltpu.VMEM((CHUNK, width), jnp.float32),
            pltpu.VMEM_SHARED((NACC, width), jnp.float32),
            pltpu.SemaphoreType.DMA,
            pltpu.SemaphoreType.DMA,
        ],
        name=f"sc_sage_agg_{width}",
    )


_TR = 2000


def _tc_body(final, sp, cp, h, wl, bl, wr, wf, bf, o):
    ssum = sp[0] + sp[1]
    cnt = cp[0][:, 0:1] + cp[1][:, 0:1]
    mean = ssum / jnp.maximum(cnt, 1.0)
    out = (jax.lax.dot_general(mean, wl[...], (((1,), (0,)), ((), ())),
                               preferred_element_type=jnp.float32)
           + bl[...]
           + jax.lax.dot_general(h[...], wr[...], (((1,), (0,)), ((), ())),
                                 preferred_element_type=jnp.float32))
    out = jnp.maximum(out, 0.0)
    if final:
        out = (jax.lax.dot_general(out, wf[...], (((1,), (0,)), ((), ())),
                                   preferred_element_type=jnp.float32)
               + bf[...])
    o[...] = out


def _tc_layer(final, sp, cp, h, wl, bl, wr, wf, bf):
    in_specs = [
        pl.BlockSpec((NC, _TR, D), lambda i: (0, i, 0)),
        pl.BlockSpec((NC, _TR, D), lambda i: (0, i, 0)),
        pl.BlockSpec((_TR, D), lambda i: (i, 0)),
        pl.BlockSpec((D, D), lambda i: (0, 0)),
        pl.BlockSpec((1, D), lambda i: (0, 0)),
        pl.BlockSpec((D, D), lambda i: (0, 0)),
        pl.BlockSpec((D, D), lambda i: (0, 0)),
        pl.BlockSpec((1, D), lambda i: (0, 0)),
    ]
    return pl.pallas_call(
        functools.partial(_tc_body, final),
        grid=(N // _TR,),
        in_specs=in_specs,
        out_specs=pl.BlockSpec((_TR, D), lambda i: (i, 0)),
        out_shape=jax.ShapeDtypeStruct((N, D), jnp.float32),
    )(sp, cp, h, wl, bl, wr, wf, bf)


def kernel(x, edge_index, batch, Wl1, bl1, Wr1, Wl2, bl2, Wr2, Wf, bf):
    del batch
    pad = EPAD - E
    src2d = jnp.concatenate(
        [edge_index[0], jnp.zeros((pad,), jnp.int32)]).reshape(-1, CHUNK)
    dst2d = jnp.concatenate(
        [edge_index[1], jnp.full((pad,), N, jnp.int32)]).reshape(-1, CHUNK)

    consts = jnp.stack([jnp.zeros((CHUNK, D), jnp.float32),
                        jnp.ones((CHUNK, D), jnp.float32)])
    s1p = _sc_agg_kernel(D)(consts, x, src2d, dst2d)
    cnt_p = _sc_count_kernel()(consts, dst2d)

    zero_w = jnp.zeros((D, D), jnp.float32)
    zero_b = jnp.zeros((1, D), jnp.float32)
    h1 = _tc_layer(False, s1p, cnt_p, x, Wl1, bl1.reshape(1, D), Wr1,
                   zero_w, zero_b)

    s2p = _sc_agg_kernel(D)(consts, h1, src2d, dst2d)
    wf_pad = jnp.zeros((D, D), jnp.float32).at[:, :C].set(Wf)
    bf_pad = jnp.zeros((1, D), jnp.float32).at[0, :C].set(bf)
    out_pad = _tc_layer(True, s2p, cnt_p, h1,
                        Wl2, bl2.reshape(1, D), Wr2, wf_pad, bf_pad)
    return out_pad[:, :C]


# speedup vs baseline: 1.0591x; 1.0591x over previous
import functools

import jax
import jax.numpy as jnp
from jax import lax
from jax.experimental import pallas as pl
from jax.experimental.pallas import tpu as pltpu
from jax.experimental.pallas import tpu_sc as plsc

N = 10000
D = 128
E = 320000
C = 40
NC = 2
NS = 16
NW = NC * NS
CHUNK = 128
EPAD = 327680
CT = EPAD // (NW * CHUNK)
HCT = 32
SLOW_CORE = 1
CT0 = 32
CT1 = 128
NACC = 10112
RT = NACC // NS
EW = 16

_QCHUNKS = [(q * CHUNK, CHUNK) for q in range(RT // CHUNK)]
if RT % CHUNK:
    _QCHUNKS.append((RT - RT % CHUNK, RT % CHUNK))


def _sc_count_body(consts, dst2d, c_out, dst_v, buf_v, cnt_sh):
    c = lax.axis_index("c")
    s = lax.axis_index("s")
    wid = c * NS + s

    pltpu.sync_copy(consts.at[0], buf_v)
    for off, rows in _QCHUNKS:
        pltpu.sync_copy(buf_v.at[pl.ds(0, rows)],
                        cnt_sh.at[pl.ds(s * RT + off, rows)])

    pltpu.sync_copy(consts.at[1], buf_v)
    plsc.subcore_barrier()

    pltpu.sync_copy(dst2d.at[pl.ds(wid * CT, CT)], dst_v)

    @pl.loop(0, CT)
    def _edge_chunk(j):
        pltpu.sync_copy(buf_v, cnt_sh.at[dst_v.at[j]], add=True)

    plsc.subcore_barrier()

    for off, rows in _QCHUNKS:
        b = s * RT + off
        pltpu.sync_copy(cnt_sh.at[pl.ds(b, rows)], buf_v.at[pl.ds(0, rows)])
        pltpu.sync_copy(buf_v.at[pl.ds(0, rows)], c_out.at[c, pl.ds(b, rows)])


@functools.lru_cache(maxsize=None)
def _sc_count_kernel():
    mesh = plsc.VectorSubcoreMesh(
        core_axis_name="c", subcore_axis_name="s",
        num_cores=NC, num_subcores=NS)
    return pl.kernel(
        _sc_count_body,
        out_type=jax.ShapeDtypeStruct((NC, NACC, D), jnp.float32),
        mesh=mesh,
        scratch_types=[
            pltpu.VMEM((CT, CHUNK), jnp.int32),
            pltpu.VMEM((CHUNK, D), jnp.float32),
            pltpu.VMEM_SHARED((NACC, D), jnp.float32),
        ],
        name="sc_sage_count",
    )


def _sc_agg_body(width, consts, table, src2d, dst2d, s_out, src_v, dst_v,
                 rows_v, rows_w, acc_sh, sem_a, sem_b):
    c = lax.axis_index("c")
    s = lax.axis_index("s")
    wid = c * NS + s

    pltpu.sync_copy(consts.at[0], rows_v)
    for off, rows in _QCHUNKS:
        pltpu.sync_copy(rows_v.at[pl.ds(0, rows)],
                        acc_sh.at[pl.ds(s * RT + off, rows)])

    plsc.subcore_barrier()

    bufs = (rows_v, rows_w)
    sems = (sem_a, sem_b)

    def _pipeline(base, n):
        pltpu.sync_copy(src2d.at[pl.ds(base, n)], src_v)
        pltpu.sync_copy(dst2d.at[pl.ds(base, n)], dst_v)
        cp = [None] * n
        cp[0] = pltpu.async_copy(table.at[src_v.at[0]], bufs[0], sems[0])
        if n > 1:
            cp[1] = pltpu.async_copy(table.at[src_v.at[1]], bufs[1], sems[1])
        for j in range(n):
            p = j % 2
            cp[j].wait()
            pltpu.sync_copy(bufs[p], acc_sh.at[dst_v.at[j]], add=True)
            if j + 2 < n:
                cp[j + 2] = pltpu.async_copy(
                    table.at[src_v.at[j + 2]], bufs[p], sems[p])

    @pl.when(c == SLOW_CORE)
    def _slow_core_edges():
        for st in range(CT0 // HCT):
            _pipeline(s * CT0 + st * HCT, HCT)

    @pl.when(c != SLOW_CORE)
    def _fast_core_edges():
        base0 = NS * CT0 + s * CT1
        for st in range(CT1 // HCT):
            _pipeline(base0 + st * HCT, HCT)

    plsc.subcore_barrier()

    for off, rows in _QCHUNKS:
        b = s * RT + off
        pltpu.sync_copy(acc_sh.at[pl.ds(b, rows)], rows_v.at[pl.ds(0, rows)])
        pltpu.sync_copy(rows_v.at[pl.ds(0, rows)], s_out.at[c, pl.ds(b, rows)])


@functools.lru_cache(maxsize=None)
def _sc_agg_kernel(width):
    mesh = plsc.VectorSubcoreMesh(
        core_axis_name="c", subcore_axis_name="s",
        num_cores=NC, num_subcores=NS)
    return pl.kernel(
        functools.partial(_sc_agg_body, width),
        out_type=jax.ShapeDtypeStruct((NC, NACC, width), jnp.float32),
        mesh=mesh,
        scratch_types=[
            pltpu.VMEM((HCT, CHUNK), jnp.int32),
            pltpu.VMEM((HCT, CHUNK), jnp.int32),
            pltpu.VMEM((CHUNK, width), jnp.float32),
            pltpu.VMEM((CHUNK, width), jnp.float32),
            pltpu.VMEM_SHARED((NACC, width), jnp.float32),
            pltpu.SemaphoreType.DMA,
            pltpu.SemaphoreType.DMA,
        ],
        name=f"sc_sage_agg_{width}",
    )


_TR = 2000


def _tc_body(final, sp, cp, h, wl, bl, wr, wf, bf, o):
    ssum = sp[0] + sp[1]
    cnt = cp[0][:, 0:1] + cp[1][:, 0:1]
    mean = ssum / jnp.maximum(cnt, 1.0)
    out = (jax.lax.dot_general(mean, wl[...], (((1,), (0,)), ((), ())),
                               preferred_element_type=jnp.float32)
           + bl[...]
           + jax.lax.dot_general(h[...], wr[...], (((1,), (0,)), ((), ())),
                                 preferred_element_type=jnp.float32))
    out = jnp.maximum(out, 0.0)
    if final:
        out = (jax.lax.dot_general(out, wf[...], (((1,), (0,)), ((), ())),
                                   preferred_element_type=jnp.float32)
               + bf[...])
    o[...] = out


def _tc_layer(final, sp, cp, h, wl, bl, wr, wf, bf):
    in_specs = [
        pl.BlockSpec((NC, _TR, D), lambda i: (0, i, 0)),
        pl.BlockSpec((NC, _TR, D), lambda i: (0, i, 0)),
        pl.BlockSpec((_TR, D), lambda i: (i, 0)),
        pl.BlockSpec((D, D), lambda i: (0, 0)),
        pl.BlockSpec((1, D), lambda i: (0, 0)),
        pl.BlockSpec((D, D), lambda i: (0, 0)),
        pl.BlockSpec((D, D), lambda i: (0, 0)),
        pl.BlockSpec((1, D), lambda i: (0, 0)),
    ]
    return pl.pallas_call(
        functools.partial(_tc_body, final),
        grid=(N // _TR,),
        in_specs=in_specs,
        out_specs=pl.BlockSpec((_TR, D), lambda i: (i, 0)),
        out_shape=jax.ShapeDtypeStruct((N, D), jnp.float32),
    )(sp, cp, h, wl, bl, wr, wf, bf)


def kernel(x, edge_index, batch, Wl1, bl1, Wr1, Wl2, bl2, Wr2, Wf, bf):
    del batch
    pad = EPAD - E
    src2d = jnp.concatenate(
        [edge_index[0], jnp.zeros((pad,), jnp.int32)]).reshape(-1, CHUNK)
    dst2d = jnp.concatenate(
        [edge_index[1], jnp.full((pad,), N, jnp.int32)]).reshape(-1, CHUNK)

    consts = jnp.stack([jnp.zeros((CHUNK, D), jnp.float32),
                        jnp.ones((CHUNK, D), jnp.float32)])
    s1p = _sc_agg_kernel(D)(consts, x, src2d, dst2d)
    cnt_p = _sc_count_kernel()(consts, dst2d)

    zero_w = jnp.zeros((D, D), jnp.float32)
    zero_b = jnp.zeros((1, D), jnp.float32)
    h1 = _tc_layer(False, s1p, cnt_p, x, Wl1, bl1.reshape(1, D), Wr1,
                   zero_w, zero_b)

    s2p = _sc_agg_kernel(D)(consts, h1, src2d, dst2d)
    wf_pad = jnp.zeros((D, D), jnp.float32).at[:, :C].set(Wf)
    bf_pad = jnp.zeros((1, D), jnp.float32).at[0, :C].set(bf)
    out_pad = _tc_layer(True, s2p, cnt_p, h1,
                        Wl2, bl2.reshape(1, D), Wr2, wf_pad, bf_pad)
    return out_pad[:, :C]


# speedup vs baseline: 1.0618x; 1.0026x over previous
import functools

import jax
import jax.numpy as jnp
from jax import lax
from jax.experimental import pallas as pl
from jax.experimental.pallas import tpu as pltpu
from jax.experimental.pallas import tpu_sc as plsc

N = 10000
D = 128
E = 320000
C = 40
NC = 2
NS = 16
NW = NC * NS
CHUNK = 128
EPAD = 327680
CT = EPAD // (NW * CHUNK)
HCT = CT // 2
NACC = 10112
RT = NACC // NS
EW = 16

_QCHUNKS = [(q * CHUNK, CHUNK) for q in range(RT // CHUNK)]
if RT % CHUNK:
    _QCHUNKS.append((RT - RT % CHUNK, RT % CHUNK))


def _sc_count_body(consts, dst2d, c_out, dst_v, buf_v, cnt_sh):
    c = lax.axis_index("c")
    s = lax.axis_index("s")
    wid = c * NS + s

    pltpu.sync_copy(consts.at[0], buf_v)
    for off, rows in _QCHUNKS:
        pltpu.sync_copy(buf_v.at[pl.ds(0, rows)],
                        cnt_sh.at[pl.ds(s * RT + off, rows)])

    pltpu.sync_copy(consts.at[1], buf_v)
    plsc.subcore_barrier()

    pltpu.sync_copy(dst2d.at[pl.ds(wid * CT, CT)], dst_v)

    @pl.loop(0, CT)
    def _edge_chunk(j):
        pltpu.sync_copy(buf_v, cnt_sh.at[dst_v.at[j]], add=True)

    plsc.subcore_barrier()

    for off, rows in _QCHUNKS:
        b = s * RT + off
        pltpu.sync_copy(cnt_sh.at[pl.ds(b, rows)], buf_v.at[pl.ds(0, rows)])
        pltpu.sync_copy(buf_v.at[pl.ds(0, rows)], c_out.at[c, pl.ds(b, rows)])


@functools.lru_cache(maxsize=None)
def _sc_count_kernel():
    mesh = plsc.VectorSubcoreMesh(
        core_axis_name="c", subcore_axis_name="s",
        num_cores=NC, num_subcores=NS)
    return pl.kernel(
        _sc_count_body,
        out_type=jax.ShapeDtypeStruct((NC, NACC, D), jnp.float32),
        mesh=mesh,
        scratch_types=[
            pltpu.VMEM((CT, CHUNK), jnp.int32),
            pltpu.VMEM((CHUNK, D), jnp.float32),
            pltpu.VMEM_SHARED((NACC, D), jnp.float32),
        ],
        name="sc_sage_count",
    )


def _sc_agg_body(width, consts, table, src2d, dst2d, s_out, src_v, dst_v,
                 rows_v, rows_w, acc_sh, sem_a, sem_b):
    c = lax.axis_index("c")
    s = lax.axis_index("s")
    wid = c * NS + s

    pltpu.sync_copy(consts.at[0], rows_v)
    for off, rows in _QCHUNKS:
        pltpu.sync_copy(rows_v.at[pl.ds(0, rows)],
                        acc_sh.at[pl.ds(s * RT + off, rows)])

    plsc.subcore_barrier()

    bufs = (rows_v, rows_w)
    sems = (sem_a, sem_b)
    for half in range(2):
        base = wid * CT + half * HCT
        pltpu.sync_copy(src2d.at[pl.ds(base, HCT)], src_v)
        pltpu.sync_copy(dst2d.at[pl.ds(base, HCT)], dst_v)
        cp = [None] * HCT
        cp[0] = pltpu.async_copy(table.at[src_v.at[0]], bufs[0], sems[0])
        cp[1] = pltpu.async_copy(table.at[src_v.at[1]], bufs[1], sems[1])
        for j in range(HCT):
            p = j % 2
            cp[j].wait()
            pltpu.sync_copy(bufs[p], acc_sh.at[dst_v.at[j]], add=True)
            if j + 2 < HCT:
                cp[j + 2] = pltpu.async_copy(
                    table.at[src_v.at[j + 2]], bufs[p], sems[p])

    plsc.subcore_barrier()

    for off, rows in _QCHUNKS:
        b = s * RT + off
        pltpu.sync_copy(acc_sh.at[pl.ds(b, rows)], rows_v.at[pl.ds(0, rows)])
        pltpu.sync_copy(rows_v.at[pl.ds(0, rows)], s_out.at[c, pl.ds(b, rows)])


@functools.lru_cache(maxsize=None)
def _sc_agg_kernel(width):
    mesh = plsc.VectorSubcoreMesh(
        core_axis_name="c", subcore_axis_name="s",
        num_cores=NC, num_subcores=NS)
    return pl.kernel(
        functools.partial(_sc_agg_body, width),
        out_type=jax.ShapeDtypeStruct((NC, NACC, width), jnp.float32),
        mesh=mesh,
        scratch_types=[
            pltpu.VMEM((HCT, CHUNK), jnp.int32),
            pltpu.VMEM((HCT, CHUNK), jnp.int32),
            pltpu.VMEM((CHUNK, width), jnp.float32),
            pltpu.VMEM((CHUNK, width), jnp.float32),
            pltpu.VMEM_SHARED((NACC, width), jnp.float32),
            pltpu.SemaphoreType.DMA,
            pltpu.SemaphoreType.DMA,
        ],
        name=f"sc_sage_agg_{width}",
    )


_TR = 2000


def _tc_body(final, sp, cp, h, wl, bl, wr, wf, bf, o):
    ssum = sp[0] + sp[1]
    cnt = cp[0][:, 0:1] + cp[1][:, 0:1]
    mean = ssum / jnp.maximum(cnt, 1.0)
    out = (jax.lax.dot_general(mean, wl[...], (((1,), (0,)), ((), ())),
                               preferred_element_type=jnp.float32)
           + bl[...]
           + jax.lax.dot_general(h[...], wr[...], (((1,), (0,)), ((), ())),
                                 preferred_element_type=jnp.float32))
    out = jnp.maximum(out, 0.0)
    if final:
        out = (jax.lax.dot_general(out, wf[...], (((1,), (0,)), ((), ())),
                                   preferred_element_type=jnp.float32)
               + bf[...])
    o[...] = out


def _tc_layer(final, sp, cp, h, wl, bl, wr, wf, bf):
    in_specs = [
        pl.BlockSpec((NC, _TR, D), lambda i: (0, i, 0)),
        pl.BlockSpec((NC, _TR, D), lambda i: (0, i, 0)),
        pl.BlockSpec((_TR, D), lambda i: (i, 0)),
        pl.BlockSpec((D, D), lambda i: (0, 0)),
        pl.BlockSpec((1, D), lambda i: (0, 0)),
        pl.BlockSpec((D, D), lambda i: (0, 0)),
        pl.BlockSpec((D, D), lambda i: (0, 0)),
        pl.BlockSpec((1, D), lambda i: (0, 0)),
    ]
    return pl.pallas_call(
        functools.partial(_tc_body, final),
        grid=(N // _TR,),
        in_specs=in_specs,
        out_specs=pl.BlockSpec((_TR, D), lambda i: (i, 0)),
        out_shape=jax.ShapeDtypeStruct((N, D), jnp.float32),
    )(sp, cp, h, wl, bl, wr, wf, bf)


def kernel(x, edge_index, batch, Wl1, bl1, Wr1, Wl2, bl2, Wr2, Wf, bf):
    del batch
    pad = EPAD - E
    src2d = jnp.concatenate(
        [edge_index[0], jnp.zeros((pad,), jnp.int32)]).reshape(-1, CHUNK)
    dst2d = jnp.concatenate(
        [edge_index[1], jnp.full((pad,), N, jnp.int32)]).reshape(-1, CHUNK)

    consts = jnp.stack([jnp.zeros((CHUNK, D), jnp.float32),
                        jnp.ones((CHUNK, D), jnp.float32)])
    s1p = _sc_agg_kernel(D)(consts, x, src2d, dst2d)
    cnt_p = _sc_count_kernel()(consts, dst2d)

    zero_w = jnp.zeros((D, D), jnp.float32)
    zero_b = jnp.zeros((1, D), jnp.float32)
    h1 = _tc_layer(False, s1p, cnt_p, x, Wl1, bl1.reshape(1, D), Wr1,
                   zero_w, zero_b)

    s2p = _sc_agg_kernel(D)(consts, h1, src2d, dst2d)
    wf_pad = jnp.zeros((D, D), jnp.float32).at[:, :C].set(Wf)
    bf_pad = jnp.zeros((1, D), jnp.float32).at[0, :C].set(bf)
    out_pad = _tc_layer(True, s2p, cnt_p, h1,
                        Wl2, bl2.reshape(1, D), Wr2, wf_pad, bf_pad)
    return out_pad[:, :C]
